# Initial kernel scaffold; baseline (speedup 1.0000x reference)
import jax
import jax.numpy as jnp
from jax.experimental import pallas as pl


def kernel(node_ids, edge_features, t, raw_memory, all_prototypes, proto_ln_g, proto_ln_b, time_w, time_b, edge_w, edge_b, query_w, query_b, cell_ln_g, cell_ln_b, gate_w, gate_b, temperature):
    raise NotImplementedError("write your pallas kernel here")


# speedup vs baseline: 1.0716x; 1.0716x over previous
import functools

import jax
import jax.numpy as jnp
from jax.experimental import pallas as pl
from jax.experimental.pallas import tpu as pltpu

_D = 128
_E = 64
_T = 64
_P = 5
_EPS = 1e-4
_M = 256
_CHUNK = 32
_CORES = 2

_F32 = jnp.float32
_HI = jax.lax.Precision.HIGHEST


def _ln(x, g, b):
    mu = jnp.mean(x, axis=-1, keepdims=True)
    xc = x - mu
    var = jnp.mean(xc * xc, axis=-1, keepdims=True)
    return xc * jax.lax.rsqrt(var + _EPS) * g + b


def _body(idx_ref, x_ref, wmat_ref, wvec_ref, protos_hbm, raw_hbm,
          out_ref, proto_buf, raw_buf, psem, rsem, *, nb):
    c = pl.program_id(0)
    i = pl.program_id(1)
    base = (c * nb + i) * _M

    def issue(ci, carry):
        for j in range(_CHUNK):
            mi = ci * _CHUNK + j
            idx = idx_ref[base + mi]
            pltpu.make_async_copy(protos_hbm.at[idx], proto_buf.at[mi],
                                  psem).start()
            pltpu.make_async_copy(raw_hbm.at[idx], raw_buf.at[mi],
                                  rsem).start()
        return carry
    jax.lax.fori_loop(0, _M // _CHUNK, issue, 0)

    pltpu.make_async_copy(proto_buf, proto_buf, psem).wait()
    pltpu.make_async_copy(raw_buf, raw_buf, rsem).wait()

    protos = proto_buf[...]
    raw = raw_buf[...].reshape(_M, _D)
    ef = x_ref[:, :_E]
    tcol = x_ref[:, _E:_E + 1]

    pg = wvec_ref[0:1, :]
    pb = wvec_ref[1:2, :]
    cg = wvec_ref[2:3, :]
    cb = wvec_ref[3:4, :]
    bq = wvec_ref[4:5, :]
    gwr = wvec_ref[5:6, :]
    gwc = wvec_ref[6:7, :]
    gwt = wvec_ref[7:8, :_T]
    tw = wvec_ref[8:9, :_T]
    tb = wvec_ref[9:10, :_T]
    gate_b = wvec_ref[10:11, 0:1]
    inv_temp = wvec_ref[10:11, 1:2]

    te = jnp.cos(tcol * tw + tb)
    pln = _ln(protos, pg, pb)

    qpre = (jnp.dot(raw, wmat_ref[0:128, :], preferred_element_type=_F32,
                    precision=_HI)
            + jnp.dot(ef, wmat_ref[128:192, :], preferred_element_type=_F32,
                      precision=_HI)
            + jnp.dot(te, wmat_ref[192:256, :], preferred_element_type=_F32,
                      precision=_HI)
            + bq)
    query = jnp.tanh(_ln(qpre, cg, cb))

    qn = query / jnp.maximum(
        jnp.sqrt(jnp.sum(query * query, axis=-1, keepdims=True)), 1e-6)
    pnorm = jnp.sqrt(jnp.sum(pln * pln, axis=-1, keepdims=True))
    pn = pln / jnp.maximum(pnorm, 1e-6)

    sim = jnp.sum(qn[:, None, :] * pn, axis=-1)
    sim = jnp.clip(sim, -30.0, 30.0) * inv_temp
    smax = jnp.max(sim, axis=-1, keepdims=True)
    ex = jnp.exp(sim - smax)
    attn = ex / jnp.sum(ex, axis=-1, keepdims=True)

    cand = jnp.clip(jnp.sum(attn[:, :, None] * pln, axis=1), -5.0, 5.0)

    gs = (jnp.sum(jnp.clip(raw, -100.0, 100.0) * gwr, axis=-1, keepdims=True)
          + jnp.sum(jnp.clip(cand, -100.0, 100.0) * gwc, axis=-1,
                    keepdims=True)
          + jnp.sum(jnp.clip(te, -100.0, 100.0) * gwt, axis=-1, keepdims=True)
          + gate_b)
    gate = jax.nn.sigmoid(gs)

    upd = (1.0 - gate) * raw + gate * cand
    out_ref[...] = jnp.clip(_ln(upd, cg, cb), -10.0, 10.0)


def kernel(node_ids, edge_features, t, raw_memory, all_prototypes,
           proto_ln_g, proto_ln_b, time_w, time_b, edge_w, edge_b,
           query_w, query_b, cell_ln_g, cell_ln_b, gate_w, gate_b,
           temperature):
    B = node_ids.shape[0]
    N, P, D = all_prototypes.shape
    E = edge_features.shape[1]
    T = time_w.shape[0]
    nb = B // _M // _CORES

    A = query_w[:, :D].T
    Bm = query_w[:, D:2 * D].T
    C = query_w[:, 2 * D:].T
    w_eq = jnp.dot(edge_w.T, Bm, precision=_HI)
    bq = query_b + jnp.dot(edge_b, Bm, precision=_HI)
    wmat = jnp.concatenate([A, w_eq, C], axis=0)

    def pad128(v):
        return jnp.pad(v, (0, D - v.shape[0]))
    inv_temp = 1.0 / (jnp.clip(temperature, 0.05, 2.0) + 1e-4)
    sc = jnp.concatenate([gate_b.astype(_F32), inv_temp.astype(_F32),
                          jnp.zeros((D - 2,), _F32)])
    z = jnp.zeros((D,), _F32)
    wvec = jnp.stack([
        proto_ln_g, proto_ln_b, cell_ln_g, cell_ln_b, bq,
        gate_w[0, :D], gate_w[0, D:2 * D], pad128(gate_w[0, 2 * D:]),
        pad128(time_w), pad128(time_b), sc, z, z, z, z, z,
    ], axis=0)

    x = jnp.pad(jnp.concatenate([edge_features, t[:, None]], axis=1),
                ((0, 0), (0, D - E - 1)))
    raw3 = raw_memory.reshape(N, 1, D)
    ids = node_ids.astype(jnp.int32)

    out = pl.pallas_call(
        functools.partial(_body, nb=nb),
        grid_spec=pltpu.PrefetchScalarGridSpec(
            num_scalar_prefetch=1,
            grid=(_CORES, nb),
            in_specs=[
                pl.BlockSpec((_M, D), lambda c, i, s: (c * nb + i, 0)),
                pl.BlockSpec((2 * D, D), lambda c, i, s: (0, 0)),
                pl.BlockSpec((16, D), lambda c, i, s: (0, 0)),
                pl.BlockSpec(memory_space=pl.ANY),
                pl.BlockSpec(memory_space=pl.ANY),
            ],
            out_specs=pl.BlockSpec((_M, D), lambda c, i, s: (c * nb + i, 0)),
            scratch_shapes=[
                pltpu.VMEM((_M, P, D), _F32),
                pltpu.VMEM((_M, 1, D), _F32),
                pltpu.SemaphoreType.DMA,
                pltpu.SemaphoreType.DMA,
            ],
        ),
        out_shape=jax.ShapeDtypeStruct((B, D), _F32),
        compiler_params=pltpu.CompilerParams(
            dimension_semantics=("parallel", "arbitrary"),
            vmem_limit_bytes=32 * 1024 * 1024,
        ),
    )(ids, x, wmat, wvec, all_prototypes, raw3)
    return out


# speedup vs baseline: 1.3102x; 1.2226x over previous
import functools

import jax
import jax.numpy as jnp
from jax.experimental import pallas as pl
from jax.experimental.pallas import tpu as pltpu

_D = 128
_E = 64
_T = 64
_P = 5
_EPS = 1e-4
_M = 256
_CORES = 2

_F32 = jnp.float32
_HI = jax.lax.Precision.HIGHEST


def _issue(idx_ref, protos_hbm, raw_hbm, pbuf, sem, base):
    for mi in range(_M):
        idx = idx_ref[base + mi]
        pltpu.make_async_copy(protos_hbm.at[idx], pbuf.at[mi, pl.ds(0, _P)],
                              sem).start()
        pltpu.make_async_copy(raw_hbm.at[idx], pbuf.at[mi, pl.ds(_P, 1)],
                              sem).start(priority=1)


def _wait(pbuf, sem):
    pltpu.make_async_copy(pbuf.at[pl.ds(0, (_M * 6) // 8)],
                          pbuf.at[pl.ds(0, (_M * 6) // 8)], sem).wait()


def _compute(pbuf, x, wmat_ref, wvec_ref):
    pg = wvec_ref[0:1, :]
    pb = wvec_ref[1:2, :]
    cg = wvec_ref[2:3, :]
    cb = wvec_ref[3:4, :]
    bq = wvec_ref[4:5, :]
    gwr = wvec_ref[5:6, :]
    gwc = wvec_ref[6:7, :]
    gwt = wvec_ref[7:8, :_T]
    tw = wvec_ref[8:9, :_T]
    tb = wvec_ref[9:10, :_T]
    gate_b = wvec_ref[10:11, 0:1]
    inv_temp = wvec_ref[10:11, 1:2]

    raw = pbuf[:, _P, :]
    ef = x[:, :_E]
    tcol = x[:, _E:_E + 1]

    te = jnp.cos(tcol * tw + tb)

    qpre = (jnp.dot(raw, wmat_ref[0:128, :], preferred_element_type=_F32,
                    precision=_HI)
            + jnp.dot(ef, wmat_ref[128:192, :], preferred_element_type=_F32,
                      precision=_HI)
            + jnp.dot(te, wmat_ref[192:256, :], preferred_element_type=_F32,
                      precision=_HI)
            + bq)
    qmu = jnp.mean(qpre, axis=-1, keepdims=True)
    qc = qpre - qmu
    qvar = jnp.mean(qc * qc, axis=-1, keepdims=True)
    query = jnp.tanh(qc * jax.lax.rsqrt(qvar + _EPS) * cg + cb)
    inv_q = jax.lax.rsqrt(
        jnp.maximum(jnp.sum(query * query, axis=-1, keepdims=True), 1e-12))

    plns = []
    sims = []
    for p in range(_P):
        y = pbuf[:, p, :]
        mu = jnp.mean(y, axis=-1, keepdims=True)
        yc = y - mu
        var = jnp.mean(yc * yc, axis=-1, keepdims=True)
        pln = yc * jax.lax.rsqrt(var + _EPS) * pg + pb
        plns.append(pln)
        inv_n = jax.lax.rsqrt(
            jnp.maximum(jnp.sum(pln * pln, axis=-1, keepdims=True), 1e-12))
        d = jnp.sum(query * pln, axis=-1, keepdims=True)
        sims.append(jnp.clip(d * inv_n * inv_q, -30.0, 30.0) * inv_temp)

    mx = jnp.maximum(jnp.maximum(jnp.maximum(sims[0], sims[1]),
                                 jnp.maximum(sims[2], sims[3])), sims[4])
    es = [jnp.exp(s - mx) for s in sims]
    inv_d = 1.0 / (es[0] + es[1] + es[2] + es[3] + es[4])
    acc = es[0] * plns[0]
    for p in range(1, _P):
        acc = acc + es[p] * plns[p]
    cand = jnp.clip(acc * inv_d, -5.0, 5.0)

    gs = (jnp.sum(jnp.clip(raw, -100.0, 100.0) * gwr, axis=-1, keepdims=True)
          + jnp.sum(jnp.clip(cand, -100.0, 100.0) * gwc, axis=-1,
                    keepdims=True)
          + jnp.sum(jnp.clip(te, -100.0, 100.0) * gwt, axis=-1, keepdims=True)
          + gate_b)
    gate = jax.nn.sigmoid(gs)

    upd = (1.0 - gate) * raw + gate * cand
    umu = jnp.mean(upd, axis=-1, keepdims=True)
    uc = upd - umu
    uvar = jnp.mean(uc * uc, axis=-1, keepdims=True)
    return jnp.clip(uc * jax.lax.rsqrt(uvar + _EPS) * cg + cb, -10.0, 10.0)


def _body(idx_ref, x_ref, wmat_ref, wvec_ref, protos_hbm, raw_hbm, out_ref,
          pbuf_a, pbuf_b, sem_a, sem_b, *, nb2, btot):
    c = pl.program_id(0)
    i = pl.program_id(1)
    base0 = (c * nb2 + i) * 2 * _M

    @pl.when(i == 0)
    def _prologue():
        _issue(idx_ref, protos_hbm, raw_hbm, pbuf_a, sem_a, base0)
        _issue(idx_ref, protos_hbm, raw_hbm, pbuf_b, sem_b, base0 + _M)

    base_a = jnp.minimum(base0 + 2 * _M, btot - _M)
    base_b = jnp.minimum(base0 + 3 * _M, btot - _M)

    _wait(pbuf_a, sem_a)
    out_ref[0:_M, :] = _compute(pbuf_a, x_ref[0:_M, :], wmat_ref, wvec_ref)
    _issue(idx_ref, protos_hbm, raw_hbm, pbuf_a, sem_a, base_a)

    _wait(pbuf_b, sem_b)
    out_ref[_M:2 * _M, :] = _compute(pbuf_b, x_ref[_M:2 * _M, :], wmat_ref,
                                     wvec_ref)
    _issue(idx_ref, protos_hbm, raw_hbm, pbuf_b, sem_b, base_b)

    @pl.when(i == nb2 - 1)
    def _drain():
        _wait(pbuf_a, sem_a)
        _wait(pbuf_b, sem_b)


def kernel(node_ids, edge_features, t, raw_memory, all_prototypes,
           proto_ln_g, proto_ln_b, time_w, time_b, edge_w, edge_b,
           query_w, query_b, cell_ln_g, cell_ln_b, gate_w, gate_b,
           temperature):
    B = node_ids.shape[0]
    N, P, D = all_prototypes.shape
    E = edge_features.shape[1]
    nb2 = B // (2 * _M) // _CORES

    A = query_w[:, :D].T
    Bm = query_w[:, D:2 * D].T
    C = query_w[:, 2 * D:].T
    w_eq = jnp.dot(edge_w.T, Bm, precision=_HI)
    bq = query_b + jnp.dot(edge_b, Bm, precision=_HI)
    wmat = jnp.concatenate([A, w_eq, C], axis=0)

    def pad128(v):
        return jnp.pad(v, (0, D - v.shape[0]))
    inv_temp = 1.0 / (jnp.clip(temperature, 0.05, 2.0) + 1e-4)
    sc = jnp.concatenate([gate_b.astype(_F32), inv_temp.astype(_F32),
                          jnp.zeros((D - 2,), _F32)])
    z = jnp.zeros((D,), _F32)
    wvec = jnp.stack([
        proto_ln_g, proto_ln_b, cell_ln_g, cell_ln_b, bq,
        gate_w[0, :D], gate_w[0, D:2 * D], pad128(gate_w[0, 2 * D:]),
        pad128(time_w), pad128(time_b), sc, z, z, z, z, z,
    ], axis=0)

    x = jnp.pad(jnp.concatenate([edge_features, t[:, None]], axis=1),
                ((0, 0), (0, D - E - 1)))
    raw3 = raw_memory.reshape(N, 1, D)
    ids = node_ids.astype(jnp.int32)

    out = pl.pallas_call(
        functools.partial(_body, nb2=nb2, btot=B),
        grid_spec=pltpu.PrefetchScalarGridSpec(
            num_scalar_prefetch=1,
            grid=(_CORES, nb2),
            in_specs=[
                pl.BlockSpec((2 * _M, D), lambda c, i, s: (c * nb2 + i, 0)),
                pl.BlockSpec((2 * D, D), lambda c, i, s: (0, 0)),
                pl.BlockSpec((16, D), lambda c, i, s: (0, 0)),
                pl.BlockSpec(memory_space=pl.ANY),
                pl.BlockSpec(memory_space=pl.ANY),
            ],
            out_specs=pl.BlockSpec((2 * _M, D),
                                   lambda c, i, s: (c * nb2 + i, 0)),
            scratch_shapes=[
                pltpu.VMEM((_M, 8, D), _F32),
                pltpu.VMEM((_M, 8, D), _F32),
                pltpu.SemaphoreType.DMA,
                pltpu.SemaphoreType.DMA,
            ],
        ),
        out_shape=jax.ShapeDtypeStruct((B, D), _F32),
        compiler_params=pltpu.CompilerParams(
            dimension_semantics=("parallel", "arbitrary"),
            vmem_limit_bytes=32 * 1024 * 1024,
        ),
    )(ids, x, wmat, wvec, all_prototypes, raw3)
    return out


# speedup vs baseline: 1.5831x; 1.2083x over previous
import functools

import jax
import jax.numpy as jnp
from jax.experimental import pallas as pl
from jax.experimental.pallas import tpu as pltpu

_D = 128
_E = 64
_T = 64
_P = 5
_EPS = 1e-4
_M = 256
_CORES = 2

_F32 = jnp.float32
_HI = jax.lax.Precision.HIGHEST

_C2PI = 0.15915494309189535
_PI_HI = 6.28125
_PI_LO = 0.0019353071795864769
_COS_C = (0.999999999882105, -0.49999999873401313, 0.041666664104076497,
          -0.0013888867374010911, 2.4800690186231253e-05,
          -2.7536982121195014e-07, 2.062070257602411e-09,
          -9.77495862031219e-12)


def _cos(x):
    k = jnp.round(x * _C2PI)
    xr = x - k * _PI_HI - k * _PI_LO
    x2 = xr * xr
    acc = jnp.float32(_COS_C[7])
    for c in _COS_C[6::-1]:
        acc = acc * x2 + jnp.float32(c)
    return acc


def _issue(idx_ref, ctab_hbm, pbuf, sem, base):
    for mi in range(_M):
        idx = idx_ref[base + mi]
        pltpu.make_async_copy(ctab_hbm.at[idx], pbuf.at[mi, pl.ds(0, 6)],
                              sem).start()


def _wait(pbuf, sem):
    pltpu.make_async_copy(pbuf.at[pl.ds(0, (_M * 6) // 8)],
                          pbuf.at[pl.ds(0, (_M * 6) // 8)], sem).wait()


def _compute(pbuf, x, wmat_ref, wvec_ref):
    pg = wvec_ref[0:1, :]
    pb = wvec_ref[1:2, :]
    cg = wvec_ref[2:3, :]
    cb = wvec_ref[3:4, :]
    bq = wvec_ref[4:5, :]
    gwr = wvec_ref[5:6, :]
    gwc = wvec_ref[6:7, :]
    gwt = wvec_ref[7:8, :_T]
    tw = wvec_ref[8:9, :_T]
    tb = wvec_ref[9:10, :_T]
    gate_b = wvec_ref[10:11, 0:1]
    inv_temp = wvec_ref[10:11, 1:2]

    raw = pbuf[:, _P, :]
    ef = x[:, :_E]
    tcol = x[:, _E:_E + 1]

    te = _cos(tcol * tw + tb)

    qpre = (jnp.dot(raw, wmat_ref[0:128, :], preferred_element_type=_F32,
                    precision=_HI)
            + jnp.dot(ef, wmat_ref[128:192, :], preferred_element_type=_F32,
                      precision=_HI)
            + jnp.dot(te, wmat_ref[192:256, :], preferred_element_type=_F32,
                      precision=_HI)
            + bq)
    qmu = jnp.mean(qpre, axis=-1, keepdims=True)
    qc = qpre - qmu
    qvar = jnp.mean(qc * qc, axis=-1, keepdims=True)
    query = jnp.tanh(qc * jax.lax.rsqrt(qvar + _EPS) * cg + cb)
    inv_q = jax.lax.rsqrt(
        jnp.maximum(jnp.sum(query * query, axis=-1, keepdims=True), 1e-12))
    qn = query * (inv_q * inv_temp)

    X2 = pbuf[...].reshape(_M * 8, _D)
    mu = jnp.mean(X2, axis=-1, keepdims=True)
    yc = X2 - mu
    var = jnp.mean(yc * yc, axis=-1, keepdims=True)
    pln = yc * jax.lax.rsqrt(var + _EPS) * pg + pb
    inv_n = jax.lax.rsqrt(
        jnp.maximum(jnp.sum(pln * pln, axis=-1, keepdims=True), 1e-12))

    qrep = jnp.broadcast_to(qn[:, None, :], (_M, 8, _D)).reshape(_M * 8, _D)
    dots = jnp.sum(qrep * pln, axis=-1, keepdims=True)
    sim = dots * inv_n

    slot = jax.lax.broadcasted_iota(jnp.int32, (_M * 8, 1), 0)
    valid = (slot & 7) < _P
    sim = jnp.where(valid, sim, -1e30)

    e3 = jnp.exp(sim).reshape(_M, 8, 1)
    att3 = e3 * (1.0 / jnp.sum(e3, axis=1, keepdims=True))
    att = att3.reshape(_M * 8, 1)
    cand = jnp.clip(jnp.sum((att * pln).reshape(_M, 8, _D), axis=1),
                    -5.0, 5.0)

    gs = (jnp.sum(jnp.clip(raw, -100.0, 100.0) * gwr, axis=-1, keepdims=True)
          + jnp.sum(jnp.clip(cand, -100.0, 100.0) * gwc, axis=-1,
                    keepdims=True)
          + jnp.sum(jnp.clip(te, -100.0, 100.0) * gwt, axis=-1, keepdims=True)
          + gate_b)
    gate = jax.nn.sigmoid(gs)

    upd = (1.0 - gate) * raw + gate * cand
    umu = jnp.mean(upd, axis=-1, keepdims=True)
    uc = upd - umu
    uvar = jnp.mean(uc * uc, axis=-1, keepdims=True)
    return jnp.clip(uc * jax.lax.rsqrt(uvar + _EPS) * cg + cb, -10.0, 10.0)


def _body(idx_ref, x_ref, wmat_ref, wvec_ref, ctab_hbm, out_ref,
          pbuf_a, pbuf_b, sem_a, sem_b, *, nb2, btot):
    c = pl.program_id(0)
    i = pl.program_id(1)
    base0 = (c * nb2 + i) * 2 * _M

    @pl.when(i == 0)
    def _prologue():
        pbuf_a[:, 6:8, :] = jnp.zeros((_M, 2, _D), _F32)
        pbuf_b[:, 6:8, :] = jnp.zeros((_M, 2, _D), _F32)
        _issue(idx_ref, ctab_hbm, pbuf_a, sem_a, base0)
        _issue(idx_ref, ctab_hbm, pbuf_b, sem_b, base0 + _M)

    base_a = jnp.minimum(base0 + 2 * _M, btot - _M)
    base_b = jnp.minimum(base0 + 3 * _M, btot - _M)

    _wait(pbuf_a, sem_a)
    _wait(pbuf_b, sem_b)
    out_ref[0:_M, :] = _compute(pbuf_a, x_ref[0:_M, :], wmat_ref, wvec_ref)
    out_ref[_M:2 * _M, :] = _compute(pbuf_b, x_ref[_M:2 * _M, :], wmat_ref,
                                     wvec_ref)
    _issue(idx_ref, ctab_hbm, pbuf_a, sem_a, base_a)
    _issue(idx_ref, ctab_hbm, pbuf_b, sem_b, base_b)

    @pl.when(i == nb2 - 1)
    def _drain():
        _wait(pbuf_a, sem_a)
        _wait(pbuf_b, sem_b)


def kernel(node_ids, edge_features, t, raw_memory, all_prototypes,
           proto_ln_g, proto_ln_b, time_w, time_b, edge_w, edge_b,
           query_w, query_b, cell_ln_g, cell_ln_b, gate_w, gate_b,
           temperature):
    B = node_ids.shape[0]
    N, P, D = all_prototypes.shape
    E = edge_features.shape[1]
    nb2 = B // (2 * _M) // _CORES

    A = query_w[:, :D].T
    Bm = query_w[:, D:2 * D].T
    C = query_w[:, 2 * D:].T
    w_eq = jnp.dot(edge_w.T, Bm, precision=_HI)
    bq = query_b + jnp.dot(edge_b, Bm, precision=_HI)
    wmat = jnp.concatenate([A, w_eq, C], axis=0)

    def pad128(v):
        return jnp.pad(v, (0, D - v.shape[0]))
    inv_temp = 1.0 / (jnp.clip(temperature, 0.05, 2.0) + 1e-4)
    sc = jnp.concatenate([gate_b.astype(_F32), inv_temp.astype(_F32),
                          jnp.zeros((D - 2,), _F32)])
    z = jnp.zeros((D,), _F32)
    wvec = jnp.stack([
        proto_ln_g, proto_ln_b, cell_ln_g, cell_ln_b, bq,
        gate_w[0, :D], gate_w[0, D:2 * D], pad128(gate_w[0, 2 * D:]),
        pad128(time_w), pad128(time_b), sc, z, z, z, z, z,
    ], axis=0)

    ctab = jnp.concatenate([all_prototypes, raw_memory[:, None, :]], axis=1)

    x = jnp.pad(jnp.concatenate([edge_features, t[:, None]], axis=1),
                ((0, 0), (0, D - E - 1)))
    ids = node_ids.astype(jnp.int32)

    out = pl.pallas_call(
        functools.partial(_body, nb2=nb2, btot=B),
        grid_spec=pltpu.PrefetchScalarGridSpec(
            num_scalar_prefetch=1,
            grid=(_CORES, nb2),
            in_specs=[
                pl.BlockSpec((2 * _M, D), lambda c, i, s: (c * nb2 + i, 0)),
                pl.BlockSpec((2 * D, D), lambda c, i, s: (0, 0)),
                pl.BlockSpec((16, D), lambda c, i, s: (0, 0)),
                pl.BlockSpec(memory_space=pl.ANY),
            ],
            out_specs=pl.BlockSpec((2 * _M, D),
                                   lambda c, i, s: (c * nb2 + i, 0)),
            scratch_shapes=[
                pltpu.VMEM((_M, 8, D), _F32),
                pltpu.VMEM((_M, 8, D), _F32),
                pltpu.SemaphoreType.DMA,
                pltpu.SemaphoreType.DMA,
            ],
        ),
        out_shape=jax.ShapeDtypeStruct((B, D), _F32),
        compiler_params=pltpu.CompilerParams(
            dimension_semantics=("parallel", "arbitrary"),
            vmem_limit_bytes=32 * 1024 * 1024,
        ),
    )(ids, x, wmat, wvec, ctab)
    return out


# speedup vs baseline: 1.8681x; 1.1800x over previous
import functools

import jax
import jax.numpy as jnp
from jax.experimental import pallas as pl
from jax.experimental.pallas import tpu as pltpu

_D = 128
_E = 64
_T = 64
_P = 5
_EPS = 1e-4
_M = 256
_CORES = 2

_F32 = jnp.float32
_HI = jax.lax.Precision.HIGHEST

_C2PI = 0.15915494309189535
_PI_HI = 6.28125
_PI_LO = 0.0019353071795864769
_COS_C = (0.999999999882105, -0.49999999873401313, 0.041666664104076497,
          -0.0013888867374010911, 2.4800690186231253e-05,
          -2.7536982121195014e-07, 2.062070257602411e-09,
          -9.77495862031219e-12)


def _cos(x):
    k = jnp.round(x * _C2PI)
    xr = x - k * _PI_HI - k * _PI_LO
    x2 = xr * xr
    acc = jnp.float32(_COS_C[7])
    for c in _COS_C[6::-1]:
        acc = acc * x2 + jnp.float32(c)
    return acc


def _issue(idx_ref, ctab_hbm, pbuf, sem, base):
    for mi in range(_M):
        idx = idx_ref[base + mi]
        pltpu.make_async_copy(ctab_hbm.at[idx], pbuf.at[mi, pl.ds(0, 6)],
                              sem).start()


def _wait(pbuf, sem):
    pltpu.make_async_copy(pbuf.at[pl.ds(0, (_M * 6) // 8)],
                          pbuf.at[pl.ds(0, (_M * 6) // 8)], sem).wait()


def _compute(pbuf, x, wmat_ref, wvec_ref):
    pg = wvec_ref[0:1, :]
    pb = wvec_ref[1:2, :]
    cg = wvec_ref[2:3, :]
    cb = wvec_ref[3:4, :]
    bq = wvec_ref[4:5, :]
    gwr = wvec_ref[5:6, :]
    gwc = wvec_ref[6:7, :]
    gwt = wvec_ref[7:8, :_T]
    tw = wvec_ref[8:9, :_T]
    tb = wvec_ref[9:10, :_T]
    gate_b = wvec_ref[10:11, 0:1]
    inv_temp = wvec_ref[10:11, 1:2]

    raw = pbuf[:, _P, :]
    ef = x[:, :_E]
    tcol = x[:, _E:_E + 1]

    te = _cos(tcol * tw + tb)

    qpre = (jnp.dot(raw, wmat_ref[0:128, :], preferred_element_type=_F32,
                    precision=_HI)
            + jnp.dot(ef, wmat_ref[128:192, :], preferred_element_type=_F32,
                      precision=_HI)
            + jnp.dot(te, wmat_ref[192:256, :], preferred_element_type=_F32,
                      precision=_HI)
            + bq)
    qmu = jnp.mean(qpre, axis=-1, keepdims=True)
    qc = qpre - qmu
    qvar = jnp.mean(qc * qc, axis=-1, keepdims=True)
    query = jnp.tanh(qc * jax.lax.rsqrt(qvar + _EPS) * cg + cb)
    inv_q = jax.lax.rsqrt(
        jnp.maximum(jnp.sum(query * query, axis=-1, keepdims=True), 1e-12))
    qn = query * (inv_q * inv_temp)

    X2 = pbuf[...].reshape(_M * 8, _D)
    mu = jnp.mean(X2, axis=-1, keepdims=True)
    yc = X2 - mu
    var = jnp.mean(yc * yc, axis=-1, keepdims=True)
    pln = yc * jax.lax.rsqrt(var + _EPS) * pg + pb
    inv_n = jax.lax.rsqrt(
        jnp.maximum(jnp.sum(pln * pln, axis=-1, keepdims=True), 1e-12))

    qrep = jnp.broadcast_to(qn[:, None, :], (_M, 8, _D)).reshape(_M * 8, _D)
    dots = jnp.sum(qrep * pln, axis=-1, keepdims=True)
    sim = dots * inv_n

    slot = jax.lax.broadcasted_iota(jnp.int32, (_M * 8, 1), 0)
    valid = (slot & 7) < _P
    sim = jnp.where(valid, sim, -1e30)

    e3 = jnp.exp(sim).reshape(_M, 8, 1)
    att3 = e3 * (1.0 / jnp.sum(e3, axis=1, keepdims=True))
    att = att3.reshape(_M * 8, 1)
    cand = jnp.clip(jnp.sum((att * pln).reshape(_M, 8, _D), axis=1),
                    -5.0, 5.0)

    gs = (jnp.sum(jnp.clip(raw, -100.0, 100.0) * gwr, axis=-1, keepdims=True)
          + jnp.sum(jnp.clip(cand, -100.0, 100.0) * gwc, axis=-1,
                    keepdims=True)
          + jnp.sum(jnp.clip(te, -100.0, 100.0) * gwt, axis=-1, keepdims=True)
          + gate_b)
    gate = jax.nn.sigmoid(gs)

    upd = (1.0 - gate) * raw + gate * cand
    umu = jnp.mean(upd, axis=-1, keepdims=True)
    uc = upd - umu
    uvar = jnp.mean(uc * uc, axis=-1, keepdims=True)
    return jnp.clip(uc * jax.lax.rsqrt(uvar + _EPS) * cg + cb, -10.0, 10.0)


def _body(idx_ref, x_ref, wmat_ref, wvec_ref, ctab_hbm, out_ref,
          pbuf_a, pbuf_b, sem_a, sem_b, *, nb2, btot):
    c = pl.program_id(0)
    i = pl.program_id(1)
    base0 = (c * nb2 + i) * 2 * _M

    @pl.when(i == 0)
    def _prologue():
        pbuf_a[:, 6:8, :] = jnp.zeros((_M, 2, _D), _F32)
        pbuf_b[:, 6:8, :] = jnp.zeros((_M, 2, _D), _F32)
        _issue(idx_ref, ctab_hbm, pbuf_a, sem_a, base0)
        _issue(idx_ref, ctab_hbm, pbuf_b, sem_b, base0 + _M)

    base_a = jnp.minimum(base0 + 2 * _M, btot - _M)
    base_b = jnp.minimum(base0 + 3 * _M, btot - _M)

    _wait(pbuf_a, sem_a)
    out_ref[0:_M, :] = _compute(pbuf_a, x_ref[0:_M, :], wmat_ref, wvec_ref)
    _issue(idx_ref, ctab_hbm, pbuf_a, sem_a, base_a)
    _wait(pbuf_b, sem_b)
    out_ref[_M:2 * _M, :] = _compute(pbuf_b, x_ref[_M:2 * _M, :], wmat_ref,
                                     wvec_ref)
    _issue(idx_ref, ctab_hbm, pbuf_b, sem_b, base_b)

    @pl.when(i == nb2 - 1)
    def _drain():
        _wait(pbuf_a, sem_a)
        _wait(pbuf_b, sem_b)


def kernel(node_ids, edge_features, t, raw_memory, all_prototypes,
           proto_ln_g, proto_ln_b, time_w, time_b, edge_w, edge_b,
           query_w, query_b, cell_ln_g, cell_ln_b, gate_w, gate_b,
           temperature):
    B = node_ids.shape[0]
    N, P, D = all_prototypes.shape
    E = edge_features.shape[1]
    nb2 = B // (2 * _M) // _CORES

    A = query_w[:, :D].T
    Bm = query_w[:, D:2 * D].T
    C = query_w[:, 2 * D:].T
    w_eq = jnp.dot(edge_w.T, Bm, precision=_HI)
    bq = query_b + jnp.dot(edge_b, Bm, precision=_HI)
    wmat = jnp.concatenate([A, w_eq, C], axis=0)

    def pad128(v):
        return jnp.pad(v, (0, D - v.shape[0]))
    inv_temp = 1.0 / (jnp.clip(temperature, 0.05, 2.0) + 1e-4)
    sc = jnp.concatenate([gate_b.astype(_F32), inv_temp.astype(_F32),
                          jnp.zeros((D - 2,), _F32)])
    z = jnp.zeros((D,), _F32)
    wvec = jnp.stack([
        proto_ln_g, proto_ln_b, cell_ln_g, cell_ln_b, bq,
        gate_w[0, :D], gate_w[0, D:2 * D], pad128(gate_w[0, 2 * D:]),
        pad128(time_w), pad128(time_b), sc, z, z, z, z, z,
    ], axis=0)

    ctab = jnp.concatenate([all_prototypes, raw_memory[:, None, :]], axis=1)

    x = jnp.pad(jnp.concatenate([edge_features, t[:, None]], axis=1),
                ((0, 0), (0, D - E - 1)))
    ids = node_ids.astype(jnp.int32)

    out = pl.pallas_call(
        functools.partial(_body, nb2=nb2, btot=B),
        grid_spec=pltpu.PrefetchScalarGridSpec(
            num_scalar_prefetch=1,
            grid=(_CORES, nb2),
            in_specs=[
                pl.BlockSpec((2 * _M, D), lambda c, i, s: (c * nb2 + i, 0)),
                pl.BlockSpec((2 * D, D), lambda c, i, s: (0, 0)),
                pl.BlockSpec((16, D), lambda c, i, s: (0, 0)),
                pl.BlockSpec(memory_space=pl.ANY),
            ],
            out_specs=pl.BlockSpec((2 * _M, D),
                                   lambda c, i, s: (c * nb2 + i, 0)),
            scratch_shapes=[
                pltpu.VMEM((_M, 8, D), _F32),
                pltpu.VMEM((_M, 8, D), _F32),
                pltpu.SemaphoreType.DMA,
                pltpu.SemaphoreType.DMA,
            ],
        ),
        out_shape=jax.ShapeDtypeStruct((B, D), _F32),
        compiler_params=pltpu.CompilerParams(
            dimension_semantics=("parallel", "arbitrary"),
            vmem_limit_bytes=32 * 1024 * 1024,
        ),
    )(ids, x, wmat, wvec, ctab)
    return out


# speedup vs baseline: 1.8762x; 1.0043x over previous
import functools

import jax
import jax.numpy as jnp
from jax.experimental import pallas as pl
from jax.experimental.pallas import tpu as pltpu

_D = 128
_E = 64
_T = 64
_P = 5
_EPS = 1e-4
_M = 256
_CORES = 2

_F32 = jnp.float32
_HI = jax.lax.Precision.HIGHEST

_C2PI = 0.15915494309189535
_PI_HI = 6.28125
_PI_LO = 0.0019353071795864769
_COS_C = (0.999999999882105, -0.49999999873401313, 0.041666664104076497,
          -0.0013888867374010911, 2.4800690186231253e-05,
          -2.7536982121195014e-07, 2.062070257602411e-09,
          -9.77495862031219e-12)


def _cos(x):
    k = jnp.round(x * _C2PI)
    xr = x - k * _PI_HI - k * _PI_LO
    x2 = xr * xr
    acc = jnp.float32(_COS_C[7])
    for c in _COS_C[6::-1]:
        acc = acc * x2 + jnp.float32(c)
    return acc


def _issue(idx_ref, ctab_hbm, pbuf, sem, base):
    for mi in range(_M):
        idx = idx_ref[base + mi]
        pltpu.make_async_copy(ctab_hbm.at[idx], pbuf.at[mi, pl.ds(0, 6)],
                              sem).start()


def _wait(pbuf, sem):
    pltpu.make_async_copy(pbuf.at[pl.ds(0, (_M * 6) // 8)],
                          pbuf.at[pl.ds(0, (_M * 6) // 8)], sem).wait()


def _compute(pbuf, x, wmat_ref, wvec_ref):
    pg = wvec_ref[0:1, :]
    pb = wvec_ref[1:2, :]
    cg = wvec_ref[2:3, :]
    cb = wvec_ref[3:4, :]
    bq = wvec_ref[4:5, :]
    gwr = wvec_ref[5:6, :]
    gwc = wvec_ref[6:7, :]
    gwt = wvec_ref[7:8, :_T]
    tw = wvec_ref[8:9, :_T]
    tb = wvec_ref[9:10, :_T]
    gate_b = wvec_ref[10:11, 0:1]
    inv_temp = wvec_ref[10:11, 1:2]

    raw = pbuf[:, _P, :]
    ef = x[:, :_E]
    tcol = x[:, _E:_E + 1]

    te = _cos(tcol * tw + tb)

    qpre = (jnp.dot(raw, wmat_ref[0:128, :], preferred_element_type=_F32,
                    precision=_HI)
            + jnp.dot(ef, wmat_ref[128:192, :], preferred_element_type=_F32,
                      precision=_HI)
            + jnp.dot(te, wmat_ref[192:256, :], preferred_element_type=_F32,
                      precision=_HI)
            + bq)
    qmu = jnp.mean(qpre, axis=-1, keepdims=True)
    qc = qpre - qmu
    qvar = jnp.mean(qc * qc, axis=-1, keepdims=True)
    query = jnp.tanh(qc * jax.lax.rsqrt(qvar + _EPS) * cg + cb)
    inv_q = jax.lax.rsqrt(
        jnp.maximum(jnp.sum(query * query, axis=-1, keepdims=True), 1e-12))
    qn = query * (inv_q * inv_temp)

    X2 = pbuf[...].reshape(_M * 8, _D)
    mu = jnp.mean(X2, axis=-1, keepdims=True)
    yc = X2 - mu
    var = jnp.mean(yc * yc, axis=-1, keepdims=True)
    pln = yc * jax.lax.rsqrt(var + _EPS) * pg + pb
    inv_n = jax.lax.rsqrt(
        jnp.maximum(jnp.sum(pln * pln, axis=-1, keepdims=True), 1e-12))

    qrep = jnp.broadcast_to(qn[:, None, :], (_M, 8, _D)).reshape(_M * 8, _D)
    dots = jnp.sum(qrep * pln, axis=-1, keepdims=True)
    sim = dots * inv_n

    slot = jax.lax.broadcasted_iota(jnp.int32, (_M * 8, 1), 0)
    valid = (slot & 7) < _P
    sim = jnp.where(valid, sim, -1e30)

    e3 = jnp.exp(sim).reshape(_M, 8, 1)
    att3 = e3 * (1.0 / jnp.sum(e3, axis=1, keepdims=True))
    att = att3.reshape(_M * 8, 1)
    cand = jnp.clip(jnp.sum((att * pln).reshape(_M, 8, _D), axis=1),
                    -5.0, 5.0)

    gs = (jnp.sum(jnp.clip(raw, -100.0, 100.0) * gwr, axis=-1, keepdims=True)
          + jnp.sum(jnp.clip(cand, -100.0, 100.0) * gwc, axis=-1,
                    keepdims=True)
          + jnp.sum(jnp.clip(te, -100.0, 100.0) * gwt, axis=-1, keepdims=True)
          + gate_b)
    gate = jax.nn.sigmoid(gs)

    upd = (1.0 - gate) * raw + gate * cand
    umu = jnp.mean(upd, axis=-1, keepdims=True)
    uc = upd - umu
    uvar = jnp.mean(uc * uc, axis=-1, keepdims=True)
    return jnp.clip(uc * jax.lax.rsqrt(uvar + _EPS) * cg + cb, -10.0, 10.0)


def _body(idx_ref, x_ref, wmat_ref, wvec_ref, ctab_hbm, out_ref,
          pb0, pb1, pb2, pb3, s0, s1, s2, s3, *, nb4, btot):
    c = pl.program_id(0)
    i = pl.program_id(1)
    base0 = (c * nb4 + i) * 4 * _M
    bufs = (pb0, pb1, pb2, pb3)
    sems = (s0, s1, s2, s3)

    @pl.when(i == 0)
    def _prologue():
        for k in range(4):
            bufs[k][:, 6:8, :] = jnp.zeros((_M, 2, _D), _F32)
            _issue(idx_ref, ctab_hbm, bufs[k], sems[k], base0 + k * _M)

    for k in range(4):
        base_n = jnp.minimum(base0 + (4 + k) * _M, btot - _M)
        _wait(bufs[k], sems[k])
        out_ref[k * _M:(k + 1) * _M, :] = _compute(
            bufs[k], x_ref[k * _M:(k + 1) * _M, :], wmat_ref, wvec_ref)
        _issue(idx_ref, ctab_hbm, bufs[k], sems[k], base_n)

    @pl.when(i == nb4 - 1)
    def _drain():
        for k in range(4):
            _wait(bufs[k], sems[k])


def kernel(node_ids, edge_features, t, raw_memory, all_prototypes,
           proto_ln_g, proto_ln_b, time_w, time_b, edge_w, edge_b,
           query_w, query_b, cell_ln_g, cell_ln_b, gate_w, gate_b,
           temperature):
    B = node_ids.shape[0]
    N, P, D = all_prototypes.shape
    E = edge_features.shape[1]
    nb4 = B // (4 * _M) // _CORES

    A = query_w[:, :D].T
    Bm = query_w[:, D:2 * D].T
    C = query_w[:, 2 * D:].T
    w_eq = jnp.dot(edge_w.T, Bm, precision=_HI)
    bq = query_b + jnp.dot(edge_b, Bm, precision=_HI)
    wmat = jnp.concatenate([A, w_eq, C], axis=0)

    def pad128(v):
        return jnp.pad(v, (0, D - v.shape[0]))
    inv_temp = 1.0 / (jnp.clip(temperature, 0.05, 2.0) + 1e-4)
    sc = jnp.concatenate([gate_b.astype(_F32), inv_temp.astype(_F32),
                          jnp.zeros((D - 2,), _F32)])
    z = jnp.zeros((D,), _F32)
    wvec = jnp.stack([
        proto_ln_g, proto_ln_b, cell_ln_g, cell_ln_b, bq,
        gate_w[0, :D], gate_w[0, D:2 * D], pad128(gate_w[0, 2 * D:]),
        pad128(time_w), pad128(time_b), sc, z, z, z, z, z,
    ], axis=0)

    ctab = jnp.concatenate([all_prototypes, raw_memory[:, None, :]], axis=1)

    x = jnp.pad(jnp.concatenate([edge_features, t[:, None]], axis=1),
                ((0, 0), (0, D - E - 1)))
    ids = node_ids.astype(jnp.int32)

    out = pl.pallas_call(
        functools.partial(_body, nb4=nb4, btot=B),
        grid_spec=pltpu.PrefetchScalarGridSpec(
            num_scalar_prefetch=1,
            grid=(_CORES, nb4),
            in_specs=[
                pl.BlockSpec((4 * _M, D), lambda c, i, s: (c * nb4 + i, 0)),
                pl.BlockSpec((2 * D, D), lambda c, i, s: (0, 0)),
                pl.BlockSpec((16, D), lambda c, i, s: (0, 0)),
                pl.BlockSpec(memory_space=pl.ANY),
            ],
            out_specs=pl.BlockSpec((4 * _M, D),
                                   lambda c, i, s: (c * nb4 + i, 0)),
            scratch_shapes=[
                pltpu.VMEM((_M, 8, D), _F32),
                pltpu.VMEM((_M, 8, D), _F32),
                pltpu.VMEM((_M, 8, D), _F32),
                pltpu.VMEM((_M, 8, D), _F32),
                pltpu.SemaphoreType.DMA,
                pltpu.SemaphoreType.DMA,
                pltpu.SemaphoreType.DMA,
                pltpu.SemaphoreType.DMA,
            ],
        ),
        out_shape=jax.ShapeDtypeStruct((B, D), _F32),
        compiler_params=pltpu.CompilerParams(
            dimension_semantics=("parallel", "arbitrary"),
            vmem_limit_bytes=32 * 1024 * 1024,
        ),
    )(ids, x, wmat, wvec, ctab)
    return out
